# Initial kernel scaffold; baseline (speedup 1.0000x reference)
import jax
import jax.numpy as jnp
from jax.experimental import pallas as pl


def kernel(coords, conns, W_in1, b_in1, W_in2, gn_in_w, gn_in_b, W_seg1, b_seg1, W_seg2, gn_seg_w, gn_seg_b, W_center, W_pre, W_suc, gn1_w, gn1_b, W_lgn, gn2_w, gn2_b):
    raise NotImplementedError("write your pallas kernel here")


# speedup vs baseline: 2.2371x; 2.2371x over previous
import functools

import jax
import jax.numpy as jnp
from jax import lax
from jax.experimental import pallas as pl
from jax.experimental.pallas import tpu as pltpu
from jax.experimental.pallas import tpu_sc as plsc

N = 10000
D = 128
NS = 6
NB = 4
E = 160000

NC = 2
NT = 16
H = D // NC
N_PAD = 10240
ROWS_PER_TILE = N_PAD // NT
EDGES_PER_TILE = E // NT
CHUNK = 80
NCHUNK = EDGES_PER_TILE // CHUNK
NPAIR = 2 * NS

def _sc_body(f_split, col0, col1, out, f_sh, macc, idx0, idx1, gbuf,
             zbuf, sem):
    c = lax.axis_index("c")
    t = lax.axis_index("s")
    r0 = t * ROWS_PER_TILE
    e0 = t * EDGES_PER_TILE

    def _zrow(i, carry):
        for j in range(H // 16):
            zbuf[i, pl.ds(j * 16, 16)] = jnp.zeros((16,), jnp.float32)
        return carry
    lax.fori_loop(0, ROWS_PER_TILE, _zrow, 0)

    pltpu.sync_copy(f_split.at[c, pl.ds(r0, ROWS_PER_TILE)],
                    f_sh.at[pl.ds(r0, ROWS_PER_TILE)])
    pltpu.sync_copy(zbuf, macc.at[pl.ds(r0, ROWS_PER_TILE)])
    plsc.subcore_barrier()

    for s in range(NS):
        for d in range(2):
            src_ref, dst_ref = (idx1, idx0) if d == 0 else (idx0, idx1)

            def _chunk(j, carry):
                base = s * E + e0 + j * CHUNK
                pltpu.sync_copy(col0.at[pl.ds(base, CHUNK)], idx0)
                pltpu.sync_copy(col1.at[pl.ds(base, CHUNK)], idx1)
                pltpu.async_copy(f_sh.at[src_ref], gbuf, sem).wait()
                pltpu.sync_copy(gbuf, macc.at[dst_ref], add=True)
                return carry
            lax.fori_loop(0, NCHUNK, _chunk, 0)
            plsc.subcore_barrier()

            p = s if d == 0 else NS + s
            pltpu.sync_copy(macc.at[pl.ds(r0, ROWS_PER_TILE)],
                            out.at[c, p, pl.ds(r0, ROWS_PER_TILE)])
            pltpu.sync_copy(zbuf, macc.at[pl.ds(r0, ROWS_PER_TILE)])
            plsc.subcore_barrier()


@functools.cache
def _make_sc_aggregate():
    mesh = plsc.VectorSubcoreMesh(core_axis_name="c", subcore_axis_name="s")
    return pl.kernel(
        _sc_body,
        out_type=jax.ShapeDtypeStruct((NC, NPAIR, N_PAD, H), jnp.float32),
        mesh=mesh,
        scratch_types=[
            pltpu.VMEM_SHARED((N_PAD, H), jnp.float32),
            pltpu.VMEM_SHARED((N_PAD, H), jnp.float32),
            pltpu.VMEM((CHUNK,), jnp.int32),
            pltpu.VMEM((CHUNK,), jnp.int32),
            pltpu.VMEM((CHUNK, H), jnp.float32),
            pltpu.VMEM((ROWS_PER_TILE, H), jnp.float32),
            pltpu.SemaphoreType.DMA,
        ],
        compiler_params=pltpu.CompilerParams(use_tc_tiling_on_sc=False),
    )


def _sc_aggregate(fsplit, col0, col1):
    return _make_sc_aggregate()(fsplit, col0, col1)


def _dot(a, b):
    return lax.dot_general(a, b, (((1,), (0,)), ((), ())),
                           preferred_element_type=jnp.float32)


def _gn(x, w, b, eps=1e-5):
    mu = jnp.mean(x, axis=-1, keepdims=True)
    var = jnp.mean((x - mu) ** 2, axis=-1, keepdims=True)
    return (x - mu) / jnp.sqrt(var + eps) * w + b


R = 1000


def _input_body(c4_ref, w1c, b1c, w2c, gwc, gbc, w1s, b1s, w2s, gws, gbs,
                ffull_ref, fsplit_ref):
    c4 = c4_ref[...]
    centers = (c4[:, 0:2] + c4[:, 2:4]) * 0.5
    diff = c4[:, 2:4] - c4[:, 0:2]

    def mlp(x, w1t, b1, w2t, gw, gb):
        h = jnp.maximum(_dot(x, w1t[...]) + b1[...], 0.0)
        h = _dot(h, w2t[...])
        return _gn(h, gw[...], gb[...])

    f = jnp.maximum(mlp(centers, w1c, b1c, w2c, gwc, gbc)
                    + mlp(diff, w1s, b1s, w2s, gws, gbs), 0.0)
    ffull_ref[...] = f
    fsplit_ref[0] = f[:, :H]
    fsplit_ref[1] = f[:, H:]


def _block_body(ff_ref, m4_ref, wct_ref, wsub_ref, g1w, g1b, wlt_ref, g2w,
                g2b, fout_ref, fsplit_ref):
    f = ff_ref[...]
    m4 = m4_ref[...]
    temp = _dot(f, wct_ref[...])
    for c in range(NC):
        for p in range(NPAIR):
            temp = temp + _dot(m4[c, p], wsub_ref[c, p])
    t = jnp.maximum(_gn(temp, g1w[...], g1b[...]), 0.0)
    t = _gn(_dot(t, wlt_ref[...]), g2w[...], g2b[...])
    out = jnp.maximum(t + f, 0.0)
    fout_ref[...] = out
    fsplit_ref[0] = out[:, :H]
    fsplit_ref[1] = out[:, H:]


def _full_spec(shape):
    nd = len(shape)
    return pl.BlockSpec(shape, lambda i, _nd=nd: (0,) * _nd)


def _input_stage(coords4, w1c, b1c, w2c, gwc, gbc, w1s, b1s, w2s, gws, gbs):
    grid = N // R
    return pl.pallas_call(
        _input_body,
        grid=(grid,),
        in_specs=[
            pl.BlockSpec((R, 4), lambda i: (i, 0)),
            _full_spec((2, D)), _full_spec((1, D)), _full_spec((D, D)),
            _full_spec((1, D)), _full_spec((1, D)),
            _full_spec((2, D)), _full_spec((1, D)), _full_spec((D, D)),
            _full_spec((1, D)), _full_spec((1, D)),
        ],
        out_specs=[
            pl.BlockSpec((R, D), lambda i: (i, 0)),
            pl.BlockSpec((NC, R, H), lambda i: (0, i, 0)),
        ],
        out_shape=[
            jax.ShapeDtypeStruct((N, D), jnp.float32),
            jax.ShapeDtypeStruct((NC, N_PAD, H), jnp.float32),
        ],
    )(coords4, w1c, b1c, w2c, gwc, gbc, w1s, b1s, w2s, gws, gbs)


def _block_stage(ffull, m4, wct, wsub, g1w, g1b, wlt, g2w, g2b):
    grid = N // R
    return pl.pallas_call(
        _block_body,
        grid=(grid,),
        in_specs=[
            pl.BlockSpec((R, D), lambda i: (i, 0)),
            pl.BlockSpec((NC, NPAIR, R, H), lambda i: (0, 0, i, 0)),
            _full_spec((D, D)),
            _full_spec((NC, NPAIR, H, D)),
            _full_spec((1, D)), _full_spec((1, D)),
            _full_spec((D, D)),
            _full_spec((1, D)), _full_spec((1, D)),
        ],
        out_specs=[
            pl.BlockSpec((R, D), lambda i: (i, 0)),
            pl.BlockSpec((NC, R, H), lambda i: (0, i, 0)),
        ],
        out_shape=[
            jax.ShapeDtypeStruct((N, D), jnp.float32),
            jax.ShapeDtypeStruct((NC, N_PAD, H), jnp.float32),
        ],
    )(ffull, m4, wct, wsub, g1w, g1b, wlt, g2w, g2b)


def kernel(coords, conns, W_in1, b_in1, W_in2, gn_in_w, gn_in_b, W_seg1,
           b_seg1, W_seg2, gn_seg_w, gn_seg_b, W_center, W_pre, W_suc,
           gn1_w, gn1_b, W_lgn, gn2_w, gn2_b):
    coords4 = coords.reshape(N, 4)
    col0 = conns[1:, :, 0].reshape(-1)
    col1 = conns[1:, :, 1].reshape(-1)
    r1 = lambda v: v.reshape(1, D)

    ffull, fsplit = _input_stage(
        coords4, W_in1.T, r1(b_in1), W_in2.T, r1(gn_in_w), r1(gn_in_b),
        W_seg1.T, r1(b_seg1), W_seg2.T, r1(gn_seg_w), r1(gn_seg_b))

    for i in range(NB):
        m4 = _sc_aggregate(fsplit, col0, col1)
        wt = jnp.swapaxes(jnp.concatenate([W_pre[i], W_suc[i]], axis=0), 1, 2)
        wsub = jnp.swapaxes(wt.reshape(NPAIR, NC, H, D), 0, 1)
        ffull, fsplit = _block_stage(
            ffull, m4, W_center[i].T, wsub, r1(gn1_w[i]), r1(gn1_b[i]),
            W_lgn[i].T, r1(gn2_w[i]), r1(gn2_b[i]))
    return ffull


# speedup vs baseline: 5.1299x; 2.2931x over previous
import functools

import jax
import jax.numpy as jnp
from jax import lax
from jax.experimental import pallas as pl
from jax.experimental.pallas import tpu as pltpu
from jax.experimental.pallas import tpu_sc as plsc

N = 10000
D = 128
NS = 6
NB = 4
E = 160000

NC = 2
NT = 16
H = D // NC
N_PAD = 10240
ROWS_PER_TILE = N_PAD // NT
CHUNK = 128
NCHUNK = 80
EPT = NCHUNK * CHUNK
E_PAD = EPT * NT
IDX_ROWS_PER_SCALE = E_PAD // CHUNK
NPAIR = 2 * NS
ZROWS = 128


def _sc_body(f_split, col0, col1, out, f_sh, macc, idx0a, idx1a,
             ga, gb, zbuf, sa, sb):
    c = lax.axis_index("c")
    t = lax.axis_index("s")
    r0 = t * ROWS_PER_TILE

    def _zrow(i, carry):
        for j in range(H // 16):
            zbuf[i, pl.ds(j * 16, 16)] = jnp.zeros((16,), jnp.float32)
        return carry
    lax.fori_loop(0, ZROWS, _zrow, 0)

    def _zero_acc():
        for k in range(ROWS_PER_TILE // ZROWS):
            pltpu.sync_copy(zbuf, macc.at[pl.ds(r0 + k * ZROWS, ZROWS)])

    pltpu.sync_copy(f_split.at[c, pl.ds(r0, ROWS_PER_TILE)],
                    f_sh.at[pl.ds(r0, ROWS_PER_TILE)])
    _zero_acc()
    plsc.subcore_barrier()

    bufs = ((ga, sa), (gb, sb))

    for s in range(NS):
        myrow = s * IDX_ROWS_PER_SCALE + t * NCHUNK
        pltpu.sync_copy(col0.at[pl.ds(myrow, NCHUNK)], idx0a)
        pltpu.sync_copy(col1.at[pl.ds(myrow, NCHUNK)], idx1a)
        for d in range(2):
            src_idx, dst_idx = (idx1a, idx0a) if d == 0 else (idx0a, idx1a)

            def _start(j, b):
                g, q = bufs[b]
                pltpu.make_async_copy(f_sh.at[src_idx.at[j]], g, q).start()

            def _wait(b):
                g, q = bufs[b]
                pltpu.make_async_copy(f_split.at[c, pl.ds(0, CHUNK)],
                                      g, q).wait()

            _start(0, 0)

            def _pair(jj, carry):
                for b in range(2):
                    cj = jj * 2 + b
                    _wait(b)

                    @pl.when(cj + 1 < NCHUNK)
                    def _():
                        _start(cj + 1, 1 - b)

                    g, _ = bufs[b]
                    pltpu.sync_copy(g, macc.at[dst_idx.at[cj]], add=True)
                return carry
            lax.fori_loop(0, NCHUNK // 2, _pair, 0)
            plsc.subcore_barrier()

            p = s if d == 0 else NS + s
            pltpu.sync_copy(macc.at[pl.ds(r0, ROWS_PER_TILE)],
                            out.at[c, p, pl.ds(r0, ROWS_PER_TILE)])
            _zero_acc()
            plsc.subcore_barrier()


@functools.cache
def _make_sc_aggregate():
    mesh = plsc.VectorSubcoreMesh(core_axis_name="c", subcore_axis_name="s")
    return pl.kernel(
        _sc_body,
        out_type=jax.ShapeDtypeStruct((NC, NPAIR, N_PAD, H), jnp.float32),
        mesh=mesh,
        scratch_types=[
            pltpu.VMEM_SHARED((N_PAD, H), jnp.float32),
            pltpu.VMEM_SHARED((N_PAD, H), jnp.float32),
            pltpu.VMEM((NCHUNK, CHUNK), jnp.int32),
            pltpu.VMEM((NCHUNK, CHUNK), jnp.int32),
            pltpu.VMEM((CHUNK, H), jnp.float32),
            pltpu.VMEM((CHUNK, H), jnp.float32),
            pltpu.VMEM((ZROWS, H), jnp.float32),
            pltpu.SemaphoreType.DMA,
            pltpu.SemaphoreType.DMA,
        ],
        compiler_params=pltpu.CompilerParams(use_tc_tiling_on_sc=False),
    )


def _sc_aggregate(fsplit, col0, col1):
    return _make_sc_aggregate()(fsplit, col0, col1)


def _dot(a, b):
    return lax.dot_general(a, b, (((1,), (0,)), ((), ())),
                           preferred_element_type=jnp.float32)


def _gn(x, w, b, eps=1e-5):
    mu = jnp.mean(x, axis=-1, keepdims=True)
    var = jnp.mean((x - mu) ** 2, axis=-1, keepdims=True)
    return (x - mu) / jnp.sqrt(var + eps) * w + b


R = 1000


def _input_body(c4_ref, w1c, b1c, w2c, gwc, gbc, w1s, b1s, w2s, gws, gbs,
                ffull_ref, fsplit_ref):
    c4 = c4_ref[...]
    centers = (c4[:, 0:2] + c4[:, 2:4]) * 0.5
    diff = c4[:, 2:4] - c4[:, 0:2]

    def mlp(x, w1t, b1, w2t, gw, gb):
        h = jnp.maximum(_dot(x, w1t[...]) + b1[...], 0.0)
        h = _dot(h, w2t[...])
        return _gn(h, gw[...], gb[...])

    f = jnp.maximum(mlp(centers, w1c, b1c, w2c, gwc, gbc)
                    + mlp(diff, w1s, b1s, w2s, gws, gbs), 0.0)
    ffull_ref[...] = f
    fsplit_ref[0] = f[:, :H]
    fsplit_ref[1] = f[:, H:]


def _block_body(ff_ref, m4_ref, wct_ref, wsub_ref, g1w, g1b, wlt_ref, g2w,
                g2b, fout_ref, fsplit_ref):
    f = ff_ref[...]
    m4 = m4_ref[...]
    temp = _dot(f, wct_ref[...])
    for c in range(NC):
        for p in range(NPAIR):
            temp = temp + _dot(m4[c, p], wsub_ref[c, p])
    t = jnp.maximum(_gn(temp, g1w[...], g1b[...]), 0.0)
    t = _gn(_dot(t, wlt_ref[...]), g2w[...], g2b[...])
    out = jnp.maximum(t + f, 0.0)
    fout_ref[...] = out
    fsplit_ref[0] = out[:, :H]
    fsplit_ref[1] = out[:, H:]


def _full_spec(shape):
    nd = len(shape)
    return pl.BlockSpec(shape, lambda i, _nd=nd: (0,) * _nd)


def _input_stage(coords4, w1c, b1c, w2c, gwc, gbc, w1s, b1s, w2s, gws, gbs):
    grid = N // R
    return pl.pallas_call(
        _input_body,
        grid=(grid,),
        in_specs=[
            pl.BlockSpec((R, 4), lambda i: (i, 0)),
            _full_spec((2, D)), _full_spec((1, D)), _full_spec((D, D)),
            _full_spec((1, D)), _full_spec((1, D)),
            _full_spec((2, D)), _full_spec((1, D)), _full_spec((D, D)),
            _full_spec((1, D)), _full_spec((1, D)),
        ],
        out_specs=[
            pl.BlockSpec((R, D), lambda i: (i, 0)),
            pl.BlockSpec((NC, R, H), lambda i: (0, i, 0)),
        ],
        out_shape=[
            jax.ShapeDtypeStruct((N, D), jnp.float32),
            jax.ShapeDtypeStruct((NC, N_PAD, H), jnp.float32),
        ],
    )(coords4, w1c, b1c, w2c, gwc, gbc, w1s, b1s, w2s, gws, gbs)


def _block_stage(ffull, m4, wct, wsub, g1w, g1b, wlt, g2w, g2b):
    grid = N // R
    return pl.pallas_call(
        _block_body,
        grid=(grid,),
        in_specs=[
            pl.BlockSpec((R, D), lambda i: (i, 0)),
            pl.BlockSpec((NC, NPAIR, R, H), lambda i: (0, 0, i, 0)),
            _full_spec((D, D)),
            _full_spec((NC, NPAIR, H, D)),
            _full_spec((1, D)), _full_spec((1, D)),
            _full_spec((D, D)),
            _full_spec((1, D)), _full_spec((1, D)),
        ],
        out_specs=[
            pl.BlockSpec((R, D), lambda i: (i, 0)),
            pl.BlockSpec((NC, R, H), lambda i: (0, i, 0)),
        ],
        out_shape=[
            jax.ShapeDtypeStruct((N, D), jnp.float32),
            jax.ShapeDtypeStruct((NC, N_PAD, H), jnp.float32),
        ],
    )(ffull, m4, wct, wsub, g1w, g1b, wlt, g2w, g2b)


def kernel(coords, conns, W_in1, b_in1, W_in2, gn_in_w, gn_in_b, W_seg1,
           b_seg1, W_seg2, gn_seg_w, gn_seg_b, W_center, W_pre, W_suc,
           gn1_w, gn1_b, W_lgn, gn2_w, gn2_b):
    coords4 = coords.reshape(N, 4)
    padv = (N + (jnp.arange(E_PAD - E, dtype=jnp.int32) % (N_PAD - N)))
    padv = jnp.broadcast_to(padv, (NS, E_PAD - E))
    col0 = jnp.concatenate([conns[1:, :, 0], padv], axis=1).reshape(-1, CHUNK)
    col1 = jnp.concatenate([conns[1:, :, 1], padv], axis=1).reshape(-1, CHUNK)
    r1 = lambda v: v.reshape(1, D)

    ffull, fsplit = _input_stage(
        coords4, W_in1.T, r1(b_in1), W_in2.T, r1(gn_in_w), r1(gn_in_b),
        W_seg1.T, r1(b_seg1), W_seg2.T, r1(gn_seg_w), r1(gn_seg_b))

    for i in range(NB):
        m4 = _sc_aggregate(fsplit, col0, col1)
        wt = jnp.swapaxes(jnp.concatenate([W_pre[i], W_suc[i]], axis=0), 1, 2)
        wsub = jnp.swapaxes(wt.reshape(NPAIR, NC, H, D), 0, 1)
        ffull, fsplit = _block_stage(
            ffull, m4, W_center[i].T, wsub, r1(gn1_w[i]), r1(gn1_b[i]),
            W_lgn[i].T, r1(gn2_w[i]), r1(gn2_b[i]))
    return ffull


# speedup vs baseline: 5.5327x; 1.0785x over previous
import functools

import jax
import jax.numpy as jnp
from jax import lax
from jax.experimental import pallas as pl
from jax.experimental.pallas import tpu as pltpu
from jax.experimental.pallas import tpu_sc as plsc

N = 10000
D = 128
NS = 6
NB = 4
E = 160000

NC = 2
NT = 16
H = D // NC
N_PAD = 10240
ROWS_PER_TILE = N_PAD // NT
CHUNK = 128
NCHUNK = 80
EPT = NCHUNK * CHUNK
E_PAD = EPT * NT
IDX_ROWS_PER_SCALE = E_PAD // CHUNK
NPAIR = 2 * NS
ZROWS = 64
IDXG = 20
NGROUP = NCHUNK // IDXG


def _sc_body(f_cat, col0, col1, col0s, col1s, out, macc0, macc1,
             idx0a, idx1a, idx0s, idx1s, gp_a, gs_a, gp_b, gs_b, zbuf,
             qp_a, qs_a, qp_b, qs_b):
    c = lax.axis_index("c")
    t = lax.axis_index("s")
    r0 = t * ROWS_PER_TILE

    def _zrow(i, carry):
        for j in range(H // 16):
            zbuf[i, pl.ds(j * 16, 16)] = jnp.zeros((16,), jnp.float32)
        return carry
    lax.fori_loop(0, ZROWS, _zrow, 0)

    def _zero_acc(acc):
        for k in range(ROWS_PER_TILE // ZROWS):
            pltpu.sync_copy(zbuf, acc.at[pl.ds(r0 + k * ZROWS, ZROWS)])

    _zero_acc(macc0)
    _zero_acc(macc1)
    plsc.subcore_barrier()

    bufs = ((gp_a, qp_a, gs_a, qs_a), (gp_b, qp_b, gs_b, qs_b))

    def _start(j, b):
        gp, qp, gs, qs = bufs[b]
        pltpu.make_async_copy(f_cat.at[idx1s.at[j]], gp, qp).start()
        pltpu.make_async_copy(f_cat.at[idx0s.at[j]], gs, qs).start()

    def _wait(b):
        gp, qp, gs, qs = bufs[b]
        pltpu.make_async_copy(f_cat.at[pl.ds(0, CHUNK)], gp, qp).wait()
        pltpu.make_async_copy(f_cat.at[pl.ds(0, CHUNK)], gs, qs).wait()

    for s in range(NS):
        myrow = s * IDX_ROWS_PER_SCALE + t * NCHUNK
        def _group(g, carry):
            grow = myrow + g * IDXG
            pltpu.sync_copy(col0.at[pl.ds(grow, IDXG)], idx0a)
            pltpu.sync_copy(col1.at[pl.ds(grow, IDXG)], idx1a)
            pltpu.sync_copy(col0s.at[c, pl.ds(grow, IDXG)], idx0s)
            pltpu.sync_copy(col1s.at[c, pl.ds(grow, IDXG)], idx1s)
            _start(0, 0)

            def _pair(jj, carry2):
                for b in range(2):
                    cj = jj * 2 + b
                    _wait(b)

                    @pl.when(cj + 1 < IDXG)
                    def _():
                        _start(cj + 1, 1 - b)

                    gp, _, gs, _ = bufs[b]
                    pltpu.sync_copy(gp, macc0.at[idx0a.at[cj]], add=True)
                    pltpu.sync_copy(gs, macc1.at[idx1a.at[cj]], add=True)
                return carry2
            lax.fori_loop(0, IDXG // 2, _pair, 0)
            return carry
        lax.fori_loop(0, NGROUP, _group, 0)
        plsc.subcore_barrier()

        pltpu.sync_copy(macc0.at[pl.ds(r0, ROWS_PER_TILE)],
                        out.at[c, s, pl.ds(r0, ROWS_PER_TILE)])
        pltpu.sync_copy(macc1.at[pl.ds(r0, ROWS_PER_TILE)],
                        out.at[c, NS + s, pl.ds(r0, ROWS_PER_TILE)])
        _zero_acc(macc0)
        _zero_acc(macc1)
        plsc.subcore_barrier()


@functools.cache
def _make_sc_aggregate():
    mesh = plsc.VectorSubcoreMesh(core_axis_name="c", subcore_axis_name="s")
    return pl.kernel(
        _sc_body,
        out_type=jax.ShapeDtypeStruct((NC, NPAIR, N_PAD, H), jnp.float32),
        mesh=mesh,
        scratch_types=[
            pltpu.VMEM_SHARED((N_PAD, H), jnp.float32),
            pltpu.VMEM_SHARED((N_PAD, H), jnp.float32),
            pltpu.VMEM((IDXG, CHUNK), jnp.int32),
            pltpu.VMEM((IDXG, CHUNK), jnp.int32),
            pltpu.VMEM((IDXG, CHUNK), jnp.int32),
            pltpu.VMEM((IDXG, CHUNK), jnp.int32),
            pltpu.VMEM((CHUNK, H), jnp.float32),
            pltpu.VMEM((CHUNK, H), jnp.float32),
            pltpu.VMEM((CHUNK, H), jnp.float32),
            pltpu.VMEM((CHUNK, H), jnp.float32),
            pltpu.VMEM((ZROWS, H), jnp.float32),
            pltpu.SemaphoreType.DMA,
            pltpu.SemaphoreType.DMA,
            pltpu.SemaphoreType.DMA,
            pltpu.SemaphoreType.DMA,
        ],
        compiler_params=pltpu.CompilerParams(use_tc_tiling_on_sc=False),
    )


def _sc_aggregate(fsplit, col0, col1, col0s, col1s):
    f_cat = fsplit.reshape(NC * N_PAD, H)
    return _make_sc_aggregate()(f_cat, col0, col1, col0s, col1s)


def _dot(a, b):
    return lax.dot_general(a, b, (((1,), (0,)), ((), ())),
                           preferred_element_type=jnp.float32)


def _gn(x, w, b, eps=1e-5):
    mu = jnp.mean(x, axis=-1, keepdims=True)
    var = jnp.mean((x - mu) ** 2, axis=-1, keepdims=True)
    return (x - mu) / jnp.sqrt(var + eps) * w + b


R = 1000


def _input_body(c4_ref, w1c, b1c, w2c, gwc, gbc, w1s, b1s, w2s, gws, gbs,
                ffull_ref, fsplit_ref):
    c4 = c4_ref[...]
    centers = (c4[:, 0:2] + c4[:, 2:4]) * 0.5
    diff = c4[:, 2:4] - c4[:, 0:2]

    def mlp(x, w1t, b1, w2t, gw, gb):
        h = jnp.maximum(_dot(x, w1t[...]) + b1[...], 0.0)
        h = _dot(h, w2t[...])
        return _gn(h, gw[...], gb[...])

    f = jnp.maximum(mlp(centers, w1c, b1c, w2c, gwc, gbc)
                    + mlp(diff, w1s, b1s, w2s, gws, gbs), 0.0)
    ffull_ref[...] = f
    fsplit_ref[0] = f[:, :H]
    fsplit_ref[1] = f[:, H:]


def _block_body(ff_ref, m4_ref, wct_ref, wsub_ref, g1w, g1b, wlt_ref, g2w,
                g2b, fout_ref, fsplit_ref):
    f = ff_ref[...]
    m4 = m4_ref[...]
    temp = _dot(f, wct_ref[...])
    for c in range(NC):
        for p in range(NPAIR):
            temp = temp + _dot(m4[c, p], wsub_ref[c, p])
    t = jnp.maximum(_gn(temp, g1w[...], g1b[...]), 0.0)
    t = _gn(_dot(t, wlt_ref[...]), g2w[...], g2b[...])
    out = jnp.maximum(t + f, 0.0)
    fout_ref[...] = out
    fsplit_ref[0] = out[:, :H]
    fsplit_ref[1] = out[:, H:]


def _full_spec(shape):
    nd = len(shape)
    return pl.BlockSpec(shape, lambda i, _nd=nd: (0,) * _nd)


def _input_stage(coords4, w1c, b1c, w2c, gwc, gbc, w1s, b1s, w2s, gws, gbs):
    grid = N // R
    return pl.pallas_call(
        _input_body,
        grid=(grid,),
        in_specs=[
            pl.BlockSpec((R, 4), lambda i: (i, 0)),
            _full_spec((2, D)), _full_spec((1, D)), _full_spec((D, D)),
            _full_spec((1, D)), _full_spec((1, D)),
            _full_spec((2, D)), _full_spec((1, D)), _full_spec((D, D)),
            _full_spec((1, D)), _full_spec((1, D)),
        ],
        out_specs=[
            pl.BlockSpec((R, D), lambda i: (i, 0)),
            pl.BlockSpec((NC, R, H), lambda i: (0, i, 0)),
        ],
        out_shape=[
            jax.ShapeDtypeStruct((N, D), jnp.float32),
            jax.ShapeDtypeStruct((NC, N_PAD, H), jnp.float32),
        ],
    )(coords4, w1c, b1c, w2c, gwc, gbc, w1s, b1s, w2s, gws, gbs)


def _block_stage(ffull, m4, wct, wsub, g1w, g1b, wlt, g2w, g2b):
    grid = N // R
    return pl.pallas_call(
        _block_body,
        grid=(grid,),
        in_specs=[
            pl.BlockSpec((R, D), lambda i: (i, 0)),
            pl.BlockSpec((NC, NPAIR, R, H), lambda i: (0, 0, i, 0)),
            _full_spec((D, D)),
            _full_spec((NC, NPAIR, H, D)),
            _full_spec((1, D)), _full_spec((1, D)),
            _full_spec((D, D)),
            _full_spec((1, D)), _full_spec((1, D)),
        ],
        out_specs=[
            pl.BlockSpec((R, D), lambda i: (i, 0)),
            pl.BlockSpec((NC, R, H), lambda i: (0, i, 0)),
        ],
        out_shape=[
            jax.ShapeDtypeStruct((N, D), jnp.float32),
            jax.ShapeDtypeStruct((NC, N_PAD, H), jnp.float32),
        ],
    )(ffull, m4, wct, wsub, g1w, g1b, wlt, g2w, g2b)


def kernel(coords, conns, W_in1, b_in1, W_in2, gn_in_w, gn_in_b, W_seg1,
           b_seg1, W_seg2, gn_seg_w, gn_seg_b, W_center, W_pre, W_suc,
           gn1_w, gn1_b, W_lgn, gn2_w, gn2_b):
    coords4 = coords.reshape(N, 4)
    padv = (N + (jnp.arange(E_PAD - E, dtype=jnp.int32) % (N_PAD - N)))
    padv = jnp.broadcast_to(padv, (NS, E_PAD - E))
    col0 = jnp.concatenate([conns[1:, :, 0], padv], axis=1).reshape(-1, CHUNK)
    col1 = jnp.concatenate([conns[1:, :, 1], padv], axis=1).reshape(-1, CHUNK)
    col0s = jnp.stack([col0, col0 + N_PAD])
    col1s = jnp.stack([col1, col1 + N_PAD])
    r1 = lambda v: v.reshape(1, D)

    ffull, fsplit = _input_stage(
        coords4, W_in1.T, r1(b_in1), W_in2.T, r1(gn_in_w), r1(gn_in_b),
        W_seg1.T, r1(b_seg1), W_seg2.T, r1(gn_seg_w), r1(gn_seg_b))

    for i in range(NB):
        m4 = _sc_aggregate(fsplit, col0, col1, col0s, col1s)
        wt = jnp.swapaxes(jnp.concatenate([W_pre[i], W_suc[i]], axis=0), 1, 2)
        wsub = jnp.swapaxes(wt.reshape(NPAIR, NC, H, D), 0, 1)
        ffull, fsplit = _block_stage(
            ffull, m4, W_center[i].T, wsub, r1(gn1_w[i]), r1(gn1_b[i]),
            W_lgn[i].T, r1(gn2_w[i]), r1(gn2_b[i]))
    return ffull


# speedup vs baseline: 5.5390x; 1.0011x over previous
import functools

import jax
import jax.numpy as jnp
from jax import lax
from jax.experimental import pallas as pl
from jax.experimental.pallas import tpu as pltpu
from jax.experimental.pallas import tpu_sc as plsc

N = 10000
D = 128
NS = 6
NB = 4
E = 160000

NC = 2
NT = 16
H = D // NC
N_PAD = 10240
ROWS_PER_TILE = N_PAD // NT
CHUNK = 128
NCHUNK = 80
EPT = NCHUNK * CHUNK
E_PAD = EPT * NT
IDX_ROWS_PER_SCALE = E_PAD // CHUNK
NPAIR = 2 * NS
ZROWS = 64
IDXG = 20
NGROUP = NCHUNK // IDXG


def _sc_body(f_cat, col0, col1, col0s, col1s, out, macc0, macc1,
             idx0a, idx1a, idx0s, idx1s, gp_a, gs_a, gp_b, gs_b, zbuf,
             qp_a, qs_a, qp_b, qs_b, rp_a, rs_a, rp_b, rs_b):
    c = lax.axis_index("c")
    t = lax.axis_index("s")
    r0 = t * ROWS_PER_TILE

    def _zrow(i, carry):
        for j in range(H // 16):
            zbuf[i, pl.ds(j * 16, 16)] = jnp.zeros((16,), jnp.float32)
        return carry
    lax.fori_loop(0, ZROWS, _zrow, 0)

    def _zero_acc(acc):
        for k in range(ROWS_PER_TILE // ZROWS):
            pltpu.sync_copy(zbuf, acc.at[pl.ds(r0 + k * ZROWS, ZROWS)])

    _zero_acc(macc0)
    _zero_acc(macc1)
    plsc.subcore_barrier()

    bufs = ((gp_a, qp_a, gs_a, qs_a, rp_a, rs_a),
            (gp_b, qp_b, gs_b, qs_b, rp_b, rs_b))

    def _start(j, b):
        gp, qp, gs, qs, _, _ = bufs[b]
        pltpu.make_async_copy(f_cat.at[idx1s.at[j]], gp, qp).start()
        pltpu.make_async_copy(f_cat.at[idx0s.at[j]], gs, qs).start()

    def _wait(b):
        gp, qp, gs, qs, _, _ = bufs[b]
        pltpu.make_async_copy(f_cat.at[pl.ds(0, CHUNK)], gp, qp).wait()
        pltpu.make_async_copy(f_cat.at[pl.ds(0, CHUNK)], gs, qs).wait()

    def _start_scatter(j, b):
        gp, _, gs, _, rp, rs = bufs[b]
        pltpu.make_async_copy(gp, macc0.at[idx0a.at[j]], rp).start(add=True)
        pltpu.make_async_copy(gs, macc1.at[idx1a.at[j]], rs).start(add=True)

    def _wait_scatter(b):
        gp, _, gs, _, rp, rs = bufs[b]
        pltpu.make_async_copy(gp, macc0.at[idx0a.at[0]], rp).wait()
        pltpu.make_async_copy(gs, macc1.at[idx1a.at[0]], rs).wait()

    for s in range(NS):
        myrow = s * IDX_ROWS_PER_SCALE + t * NCHUNK
        def _group(g, carry):
            grow = myrow + g * IDXG
            pltpu.sync_copy(col0.at[pl.ds(grow, IDXG)], idx0a)
            pltpu.sync_copy(col1.at[pl.ds(grow, IDXG)], idx1a)
            pltpu.sync_copy(col0s.at[c, pl.ds(grow, IDXG)], idx0s)
            pltpu.sync_copy(col1s.at[c, pl.ds(grow, IDXG)], idx1s)
            _start(0, 0)

            def _pair(jj, carry2):
                for b in range(2):
                    cj = jj * 2 + b
                    _wait(b)

                    @pl.when(cj >= 1)
                    def _():
                        _wait_scatter(1 - b)

                    @pl.when(cj + 1 < IDXG)
                    def _():
                        _start(cj + 1, 1 - b)

                    _start_scatter(cj, b)
                return carry2
            lax.fori_loop(0, IDXG // 2, _pair, 0)
            _wait_scatter(1)
            return carry
        lax.fori_loop(0, NGROUP, _group, 0)
        plsc.subcore_barrier()

        pltpu.sync_copy(macc0.at[pl.ds(r0, ROWS_PER_TILE)],
                        out.at[c, s, pl.ds(r0, ROWS_PER_TILE)])
        pltpu.sync_copy(macc1.at[pl.ds(r0, ROWS_PER_TILE)],
                        out.at[c, NS + s, pl.ds(r0, ROWS_PER_TILE)])
        _zero_acc(macc0)
        _zero_acc(macc1)
        plsc.subcore_barrier()


@functools.cache
def _make_sc_aggregate():
    mesh = plsc.VectorSubcoreMesh(core_axis_name="c", subcore_axis_name="s")
    return pl.kernel(
        _sc_body,
        out_type=jax.ShapeDtypeStruct((NC, NPAIR, N_PAD, H), jnp.float32),
        mesh=mesh,
        scratch_types=[
            pltpu.VMEM_SHARED((N_PAD, H), jnp.float32),
            pltpu.VMEM_SHARED((N_PAD, H), jnp.float32),
            pltpu.VMEM((IDXG, CHUNK), jnp.int32),
            pltpu.VMEM((IDXG, CHUNK), jnp.int32),
            pltpu.VMEM((IDXG, CHUNK), jnp.int32),
            pltpu.VMEM((IDXG, CHUNK), jnp.int32),
            pltpu.VMEM((CHUNK, H), jnp.float32),
            pltpu.VMEM((CHUNK, H), jnp.float32),
            pltpu.VMEM((CHUNK, H), jnp.float32),
            pltpu.VMEM((CHUNK, H), jnp.float32),
            pltpu.VMEM((ZROWS, H), jnp.float32),
            pltpu.SemaphoreType.DMA,
            pltpu.SemaphoreType.DMA,
            pltpu.SemaphoreType.DMA,
            pltpu.SemaphoreType.DMA,
            pltpu.SemaphoreType.DMA,
            pltpu.SemaphoreType.DMA,
            pltpu.SemaphoreType.DMA,
            pltpu.SemaphoreType.DMA,
        ],
        compiler_params=pltpu.CompilerParams(use_tc_tiling_on_sc=False),
    )


def _sc_aggregate(fsplit, col0, col1, col0s, col1s):
    f_cat = fsplit.reshape(NC * N_PAD, H)
    return _make_sc_aggregate()(f_cat, col0, col1, col0s, col1s)


def _dot(a, b):
    return lax.dot_general(a, b, (((1,), (0,)), ((), ())),
                           preferred_element_type=jnp.float32)


def _gn(x, w, b, eps=1e-5):
    mu = jnp.mean(x, axis=-1, keepdims=True)
    var = jnp.mean((x - mu) ** 2, axis=-1, keepdims=True)
    return (x - mu) / jnp.sqrt(var + eps) * w + b


R = 1000


def _input_body(c4_ref, w1c, b1c, w2c, gwc, gbc, w1s, b1s, w2s, gws, gbs,
                ffull_ref, fsplit_ref):
    c4 = c4_ref[...]
    centers = (c4[:, 0:2] + c4[:, 2:4]) * 0.5
    diff = c4[:, 2:4] - c4[:, 0:2]

    def mlp(x, w1t, b1, w2t, gw, gb):
        h = jnp.maximum(_dot(x, w1t[...]) + b1[...], 0.0)
        h = _dot(h, w2t[...])
        return _gn(h, gw[...], gb[...])

    f = jnp.maximum(mlp(centers, w1c, b1c, w2c, gwc, gbc)
                    + mlp(diff, w1s, b1s, w2s, gws, gbs), 0.0)
    ffull_ref[...] = f
    fsplit_ref[0] = f[:, :H]
    fsplit_ref[1] = f[:, H:]


def _block_body(ff_ref, m4_ref, wct_ref, wsub_ref, g1w, g1b, wlt_ref, g2w,
                g2b, fout_ref, fsplit_ref):
    f = ff_ref[...]
    m4 = m4_ref[...]
    temp = _dot(f, wct_ref[...])
    for c in range(NC):
        for p in range(NPAIR):
            temp = temp + _dot(m4[c, p], wsub_ref[c, p])
    t = jnp.maximum(_gn(temp, g1w[...], g1b[...]), 0.0)
    t = _gn(_dot(t, wlt_ref[...]), g2w[...], g2b[...])
    out = jnp.maximum(t + f, 0.0)
    fout_ref[...] = out
    fsplit_ref[0] = out[:, :H]
    fsplit_ref[1] = out[:, H:]


def _full_spec(shape):
    nd = len(shape)
    return pl.BlockSpec(shape, lambda i, _nd=nd: (0,) * _nd)


def _input_stage(coords4, w1c, b1c, w2c, gwc, gbc, w1s, b1s, w2s, gws, gbs):
    grid = N // R
    return pl.pallas_call(
        _input_body,
        grid=(grid,),
        in_specs=[
            pl.BlockSpec((R, 4), lambda i: (i, 0)),
            _full_spec((2, D)), _full_spec((1, D)), _full_spec((D, D)),
            _full_spec((1, D)), _full_spec((1, D)),
            _full_spec((2, D)), _full_spec((1, D)), _full_spec((D, D)),
            _full_spec((1, D)), _full_spec((1, D)),
        ],
        out_specs=[
            pl.BlockSpec((R, D), lambda i: (i, 0)),
            pl.BlockSpec((NC, R, H), lambda i: (0, i, 0)),
        ],
        out_shape=[
            jax.ShapeDtypeStruct((N, D), jnp.float32),
            jax.ShapeDtypeStruct((NC, N_PAD, H), jnp.float32),
        ],
    )(coords4, w1c, b1c, w2c, gwc, gbc, w1s, b1s, w2s, gws, gbs)


def _block_stage(ffull, m4, wct, wsub, g1w, g1b, wlt, g2w, g2b):
    grid = N // R
    return pl.pallas_call(
        _block_body,
        grid=(grid,),
        in_specs=[
            pl.BlockSpec((R, D), lambda i: (i, 0)),
            pl.BlockSpec((NC, NPAIR, R, H), lambda i: (0, 0, i, 0)),
            _full_spec((D, D)),
            _full_spec((NC, NPAIR, H, D)),
            _full_spec((1, D)), _full_spec((1, D)),
            _full_spec((D, D)),
            _full_spec((1, D)), _full_spec((1, D)),
        ],
        out_specs=[
            pl.BlockSpec((R, D), lambda i: (i, 0)),
            pl.BlockSpec((NC, R, H), lambda i: (0, i, 0)),
        ],
        out_shape=[
            jax.ShapeDtypeStruct((N, D), jnp.float32),
            jax.ShapeDtypeStruct((NC, N_PAD, H), jnp.float32),
        ],
    )(ffull, m4, wct, wsub, g1w, g1b, wlt, g2w, g2b)


def kernel(coords, conns, W_in1, b_in1, W_in2, gn_in_w, gn_in_b, W_seg1,
           b_seg1, W_seg2, gn_seg_w, gn_seg_b, W_center, W_pre, W_suc,
           gn1_w, gn1_b, W_lgn, gn2_w, gn2_b):
    coords4 = coords.reshape(N, 4)
    padv = (N + (jnp.arange(E_PAD - E, dtype=jnp.int32) % (N_PAD - N)))
    padv = jnp.broadcast_to(padv, (NS, E_PAD - E))
    col0 = jnp.concatenate([conns[1:, :, 0], padv], axis=1).reshape(-1, CHUNK)
    col1 = jnp.concatenate([conns[1:, :, 1], padv], axis=1).reshape(-1, CHUNK)
    col0s = jnp.stack([col0, col0 + N_PAD])
    col1s = jnp.stack([col1, col1 + N_PAD])
    r1 = lambda v: v.reshape(1, D)

    ffull, fsplit = _input_stage(
        coords4, W_in1.T, r1(b_in1), W_in2.T, r1(gn_in_w), r1(gn_in_b),
        W_seg1.T, r1(b_seg1), W_seg2.T, r1(gn_seg_w), r1(gn_seg_b))

    for i in range(NB):
        m4 = _sc_aggregate(fsplit, col0, col1, col0s, col1s)
        wt = jnp.swapaxes(jnp.concatenate([W_pre[i], W_suc[i]], axis=0), 1, 2)
        wsub = jnp.swapaxes(wt.reshape(NPAIR, NC, H, D), 0, 1)
        ffull, fsplit = _block_stage(
            ffull, m4, W_center[i].T, wsub, r1(gn1_w[i]), r1(gn1_b[i]),
            W_lgn[i].T, r1(gn2_w[i]), r1(gn2_b[i]))
    return ffull


# speedup vs baseline: 5.7682x; 1.0414x over previous
import functools

import jax
import jax.numpy as jnp
from jax import lax
from jax.experimental import pallas as pl
from jax.experimental.pallas import tpu as pltpu
from jax.experimental.pallas import tpu_sc as plsc

N = 10000
D = 128
NS = 6
NB = 4
E = 160000

NC = 2
NT = 16
H = D // NC
N_PAD = 10240
ROWS_PER_TILE = N_PAD // NT
CHUNK = 128
NCHUNK = 80
EPT = NCHUNK * CHUNK
E_PAD = EPT * NT
IDX_ROWS_PER_SCALE = E_PAD // CHUNK
NPAIR = 2 * NS
ZROWS = 64
IDXG = 20
NGROUP = NCHUNK // IDXG


def _sc_body(f_cat, col0, col1, col0s, col1s, out, macc0, macc1,
             idx0a, idx1a, idx0s, idx1s, gp_a, gs_a, gp_b, gs_b, zbuf,
             qp_a, qs_a, qp_b, qs_b, rp_a, rs_a, rp_b, rs_b):
    c = lax.axis_index("c")
    t = lax.axis_index("s")
    r0 = t * ROWS_PER_TILE

    def _zrow(i, carry):
        for j in range(H // 16):
            zbuf[i, pl.ds(j * 16, 16)] = jnp.zeros((16,), jnp.float32)
        return carry
    lax.fori_loop(0, ZROWS, _zrow, 0)

    def _zero_acc(acc):
        for k in range(ROWS_PER_TILE // ZROWS):
            pltpu.sync_copy(zbuf, acc.at[pl.ds(r0 + k * ZROWS, ZROWS)])

    _zero_acc(macc0)
    _zero_acc(macc1)
    plsc.subcore_barrier()

    bufs = ((gp_a, qp_a, gs_a, qs_a, rp_a, rs_a),
            (gp_b, qp_b, gs_b, qs_b, rp_b, rs_b))

    def _start(j, b):
        gp, qp, gs, qs, _, _ = bufs[b]
        pltpu.make_async_copy(f_cat.at[idx1s.at[j]], gp, qp).start()
        pltpu.make_async_copy(f_cat.at[idx0s.at[j]], gs, qs).start()

    def _wait(b):
        gp, qp, gs, qs, _, _ = bufs[b]
        pltpu.make_async_copy(f_cat.at[pl.ds(0, CHUNK)], gp, qp).wait()
        pltpu.make_async_copy(f_cat.at[pl.ds(0, CHUNK)], gs, qs).wait()

    def _start_scatter(j, b):
        gp, _, gs, _, rp, rs = bufs[b]
        pltpu.make_async_copy(gp, macc0.at[idx0a.at[j]], rp).start(add=True)
        pltpu.make_async_copy(gs, macc1.at[idx1a.at[j]], rs).start(add=True)

    def _wait_scatter(b):
        gp, _, gs, _, rp, rs = bufs[b]
        pltpu.make_async_copy(gp, macc0.at[idx0a.at[0]], rp).wait()
        pltpu.make_async_copy(gs, macc1.at[idx1a.at[0]], rs).wait()

    for s in range(NS):
        myrow = s * IDX_ROWS_PER_SCALE + t * NCHUNK
        def _group(g, carry):
            grow = myrow + g * IDXG
            pltpu.make_async_copy(col0.at[pl.ds(grow, IDXG)], idx0a,
                                  qp_a).start()
            pltpu.make_async_copy(col1.at[pl.ds(grow, IDXG)], idx1a,
                                  qs_a).start()
            pltpu.make_async_copy(col0s.at[c, pl.ds(grow, IDXG)], idx0s,
                                  qp_b).start()
            pltpu.make_async_copy(col1s.at[c, pl.ds(grow, IDXG)], idx1s,
                                  qs_b).start()
            pltpu.make_async_copy(col0.at[pl.ds(grow, IDXG)], idx0a,
                                  qp_a).wait()
            pltpu.make_async_copy(col1.at[pl.ds(grow, IDXG)], idx1a,
                                  qs_a).wait()
            pltpu.make_async_copy(col0s.at[c, pl.ds(grow, IDXG)], idx0s,
                                  qp_b).wait()
            pltpu.make_async_copy(col1s.at[c, pl.ds(grow, IDXG)], idx1s,
                                  qs_b).wait()
            _start(0, 0)

            def _pair(jj, carry2):
                for b in range(2):
                    cj = jj * 2 + b
                    _wait(b)

                    @pl.when(cj >= 1)
                    def _():
                        _wait_scatter(1 - b)

                    @pl.when(cj + 1 < IDXG)
                    def _():
                        _start(cj + 1, 1 - b)

                    _start_scatter(cj, b)
                return carry2
            lax.fori_loop(0, IDXG // 2, _pair, 0)
            _wait_scatter(1)
            return carry
        lax.fori_loop(0, NGROUP, _group, 0)
        plsc.subcore_barrier()

        pltpu.make_async_copy(macc0.at[pl.ds(r0, ROWS_PER_TILE)],
                              out.at[c, s, pl.ds(r0, ROWS_PER_TILE)],
                              rp_a).start()
        pltpu.make_async_copy(macc1.at[pl.ds(r0, ROWS_PER_TILE)],
                              out.at[c, NS + s, pl.ds(r0, ROWS_PER_TILE)],
                              rs_a).start()
        pltpu.make_async_copy(macc0.at[pl.ds(r0, ROWS_PER_TILE)],
                              out.at[c, s, pl.ds(r0, ROWS_PER_TILE)],
                              rp_a).wait()
        pltpu.make_async_copy(macc1.at[pl.ds(r0, ROWS_PER_TILE)],
                              out.at[c, NS + s, pl.ds(r0, ROWS_PER_TILE)],
                              rs_a).wait()
        for k in range(ROWS_PER_TILE // ZROWS):
            pltpu.make_async_copy(
                zbuf, macc0.at[pl.ds(r0 + k * ZROWS, ZROWS)], rp_b).start()
            pltpu.make_async_copy(
                zbuf, macc1.at[pl.ds(r0 + k * ZROWS, ZROWS)], rs_b).start()
        for k in range(ROWS_PER_TILE // ZROWS):
            pltpu.make_async_copy(
                zbuf, macc0.at[pl.ds(r0 + k * ZROWS, ZROWS)], rp_b).wait()
            pltpu.make_async_copy(
                zbuf, macc1.at[pl.ds(r0 + k * ZROWS, ZROWS)], rs_b).wait()
        plsc.subcore_barrier()


@functools.cache
def _make_sc_aggregate():
    mesh = plsc.VectorSubcoreMesh(core_axis_name="c", subcore_axis_name="s")
    return pl.kernel(
        _sc_body,
        out_type=jax.ShapeDtypeStruct((NC, NPAIR, N_PAD, H), jnp.float32),
        mesh=mesh,
        scratch_types=[
            pltpu.VMEM_SHARED((N_PAD, H), jnp.float32),
            pltpu.VMEM_SHARED((N_PAD, H), jnp.float32),
            pltpu.VMEM((IDXG, CHUNK), jnp.int32),
            pltpu.VMEM((IDXG, CHUNK), jnp.int32),
            pltpu.VMEM((IDXG, CHUNK), jnp.int32),
            pltpu.VMEM((IDXG, CHUNK), jnp.int32),
            pltpu.VMEM((CHUNK, H), jnp.float32),
            pltpu.VMEM((CHUNK, H), jnp.float32),
            pltpu.VMEM((CHUNK, H), jnp.float32),
            pltpu.VMEM((CHUNK, H), jnp.float32),
            pltpu.VMEM((ZROWS, H), jnp.float32),
            pltpu.SemaphoreType.DMA,
            pltpu.SemaphoreType.DMA,
            pltpu.SemaphoreType.DMA,
            pltpu.SemaphoreType.DMA,
            pltpu.SemaphoreType.DMA,
            pltpu.SemaphoreType.DMA,
            pltpu.SemaphoreType.DMA,
            pltpu.SemaphoreType.DMA,
        ],
        compiler_params=pltpu.CompilerParams(use_tc_tiling_on_sc=False),
    )


def _sc_aggregate(fsplit, col0, col1, col0s, col1s):
    f_cat = fsplit.reshape(NC * N_PAD, H)
    return _make_sc_aggregate()(f_cat, col0, col1, col0s, col1s)


def _dot(a, b):
    return lax.dot_general(a, b, (((1,), (0,)), ((), ())),
                           preferred_element_type=jnp.float32)


def _gn(x, w, b, eps=1e-5):
    mu = jnp.mean(x, axis=-1, keepdims=True)
    var = jnp.mean((x - mu) ** 2, axis=-1, keepdims=True)
    return (x - mu) / jnp.sqrt(var + eps) * w + b


R = 1000


def _input_body(c4_ref, w1c, b1c, w2c, gwc, gbc, w1s, b1s, w2s, gws, gbs,
                ffull_ref, fsplit_ref):
    c4 = c4_ref[...]
    centers = (c4[:, 0:2] + c4[:, 2:4]) * 0.5
    diff = c4[:, 2:4] - c4[:, 0:2]

    def mlp(x, w1t, b1, w2t, gw, gb):
        h = jnp.maximum(_dot(x, w1t[...]) + b1[...], 0.0)
        h = _dot(h, w2t[...])
        return _gn(h, gw[...], gb[...])

    f = jnp.maximum(mlp(centers, w1c, b1c, w2c, gwc, gbc)
                    + mlp(diff, w1s, b1s, w2s, gws, gbs), 0.0)
    ffull_ref[...] = f
    fsplit_ref[0] = f[:, :H]
    fsplit_ref[1] = f[:, H:]


def _block_body(ff_ref, m4_ref, wct_ref, wsub_ref, g1w, g1b, wlt_ref, g2w,
                g2b, fout_ref, fsplit_ref):
    f = ff_ref[...]
    m4 = m4_ref[...]
    temp = _dot(f, wct_ref[...])
    for c in range(NC):
        for p in range(NPAIR):
            temp = temp + _dot(m4[c, p], wsub_ref[c, p])
    t = jnp.maximum(_gn(temp, g1w[...], g1b[...]), 0.0)
    t = _gn(_dot(t, wlt_ref[...]), g2w[...], g2b[...])
    out = jnp.maximum(t + f, 0.0)
    fout_ref[...] = out
    fsplit_ref[0] = out[:, :H]
    fsplit_ref[1] = out[:, H:]


def _full_spec(shape):
    nd = len(shape)
    return pl.BlockSpec(shape, lambda i, _nd=nd: (0,) * _nd)


def _input_stage(coords4, w1c, b1c, w2c, gwc, gbc, w1s, b1s, w2s, gws, gbs):
    grid = N // R
    return pl.pallas_call(
        _input_body,
        grid=(grid,),
        in_specs=[
            pl.BlockSpec((R, 4), lambda i: (i, 0)),
            _full_spec((2, D)), _full_spec((1, D)), _full_spec((D, D)),
            _full_spec((1, D)), _full_spec((1, D)),
            _full_spec((2, D)), _full_spec((1, D)), _full_spec((D, D)),
            _full_spec((1, D)), _full_spec((1, D)),
        ],
        out_specs=[
            pl.BlockSpec((R, D), lambda i: (i, 0)),
            pl.BlockSpec((NC, R, H), lambda i: (0, i, 0)),
        ],
        out_shape=[
            jax.ShapeDtypeStruct((N, D), jnp.float32),
            jax.ShapeDtypeStruct((NC, N_PAD, H), jnp.float32),
        ],
    )(coords4, w1c, b1c, w2c, gwc, gbc, w1s, b1s, w2s, gws, gbs)


def _block_stage(ffull, m4, wct, wsub, g1w, g1b, wlt, g2w, g2b):
    grid = N // R
    return pl.pallas_call(
        _block_body,
        grid=(grid,),
        in_specs=[
            pl.BlockSpec((R, D), lambda i: (i, 0)),
            pl.BlockSpec((NC, NPAIR, R, H), lambda i: (0, 0, i, 0)),
            _full_spec((D, D)),
            _full_spec((NC, NPAIR, H, D)),
            _full_spec((1, D)), _full_spec((1, D)),
            _full_spec((D, D)),
            _full_spec((1, D)), _full_spec((1, D)),
        ],
        out_specs=[
            pl.BlockSpec((R, D), lambda i: (i, 0)),
            pl.BlockSpec((NC, R, H), lambda i: (0, i, 0)),
        ],
        out_shape=[
            jax.ShapeDtypeStruct((N, D), jnp.float32),
            jax.ShapeDtypeStruct((NC, N_PAD, H), jnp.float32),
        ],
    )(ffull, m4, wct, wsub, g1w, g1b, wlt, g2w, g2b)


def kernel(coords, conns, W_in1, b_in1, W_in2, gn_in_w, gn_in_b, W_seg1,
           b_seg1, W_seg2, gn_seg_w, gn_seg_b, W_center, W_pre, W_suc,
           gn1_w, gn1_b, W_lgn, gn2_w, gn2_b):
    coords4 = coords.reshape(N, 4)
    padv = (N + (jnp.arange(E_PAD - E, dtype=jnp.int32) % (N_PAD - N)))
    padv = jnp.broadcast_to(padv, (NS, E_PAD - E))
    col0 = jnp.concatenate([conns[1:, :, 0], padv], axis=1).reshape(-1, CHUNK)
    col1 = jnp.concatenate([conns[1:, :, 1], padv], axis=1).reshape(-1, CHUNK)
    col0s = jnp.stack([col0, col0 + N_PAD])
    col1s = jnp.stack([col1, col1 + N_PAD])
    r1 = lambda v: v.reshape(1, D)

    ffull, fsplit = _input_stage(
        coords4, W_in1.T, r1(b_in1), W_in2.T, r1(gn_in_w), r1(gn_in_b),
        W_seg1.T, r1(b_seg1), W_seg2.T, r1(gn_seg_w), r1(gn_seg_b))

    for i in range(NB):
        m4 = _sc_aggregate(fsplit, col0, col1, col0s, col1s)
        wt = jnp.swapaxes(jnp.concatenate([W_pre[i], W_suc[i]], axis=0), 1, 2)
        wsub = jnp.swapaxes(wt.reshape(NPAIR, NC, H, D), 0, 1)
        ffull, fsplit = _block_stage(
            ffull, m4, W_center[i].T, wsub, r1(gn1_w[i]), r1(gn1_b[i]),
            W_lgn[i].T, r1(gn2_w[i]), r1(gn2_b[i]))
    return ffull


# speedup vs baseline: 5.8795x; 1.0193x over previous
import functools

import jax
import jax.numpy as jnp
from jax import lax
from jax.experimental import pallas as pl
from jax.experimental.pallas import tpu as pltpu
from jax.experimental.pallas import tpu_sc as plsc

N = 10000
D = 128
NS = 6
NB = 4
E = 160000

NC = 2
NT = 16
H = D // NC
N_PAD = 10240
ROWS_PER_TILE = N_PAD // NT
CHUNK = 128
NCHUNK = 80
EPT = NCHUNK * CHUNK
E_PAD = EPT * NT
IDX_ROWS_PER_SCALE = E_PAD // CHUNK
NPAIR = 2 * NS
ZROWS = 64
IDXG = 20
NGROUP = NCHUNK // IDXG


def _sc_body(f_cat, col0, col1, col0s, col1s, out, macc0, macc1,
             idx0a, idx1a, idx0s, idx1s, gp_a, gs_a, gp_b, gs_b, zbuf,
             qp_a, qs_a, qp_b, qs_b, rp_a, rs_a, rp_b, rs_b):
    c = lax.axis_index("c")
    t = lax.axis_index("s")
    r0 = t * ROWS_PER_TILE

    def _zrow(i, carry):
        for j in range(H // 16):
            zbuf[i, pl.ds(j * 16, 16)] = jnp.zeros((16,), jnp.float32)
        return carry
    lax.fori_loop(0, ZROWS, _zrow, 0)

    def _zero_acc(acc):
        for k in range(ROWS_PER_TILE // ZROWS):
            pltpu.sync_copy(zbuf, acc.at[pl.ds(r0 + k * ZROWS, ZROWS)])

    _zero_acc(macc0)
    _zero_acc(macc1)
    plsc.subcore_barrier()

    bufs = ((gp_a, qp_a, gs_a, qs_a, rp_a, rs_a),
            (gp_b, qp_b, gs_b, qs_b, rp_b, rs_b))

    def _start(j, b):
        gp, qp, gs, qs, _, _ = bufs[b]
        pltpu.make_async_copy(f_cat.at[idx1s.at[j]], gp, qp).start()
        pltpu.make_async_copy(f_cat.at[idx0s.at[j]], gs, qs).start()

    def _wait(b):
        gp, qp, gs, qs, _, _ = bufs[b]
        pltpu.make_async_copy(f_cat.at[pl.ds(0, CHUNK)], gp, qp).wait()
        pltpu.make_async_copy(f_cat.at[pl.ds(0, CHUNK)], gs, qs).wait()

    def _start_scatter(j, b):
        gp, _, gs, _, rp, rs = bufs[b]
        pltpu.make_async_copy(gp, macc0.at[idx0a.at[j]], rp).start(add=True)
        pltpu.make_async_copy(gs, macc1.at[idx1a.at[j]], rs).start(add=True)

    def _wait_scatter(b):
        gp, _, gs, _, rp, rs = bufs[b]
        pltpu.make_async_copy(gp, macc0.at[idx0a.at[0]], rp).wait()
        pltpu.make_async_copy(gs, macc1.at[idx1a.at[0]], rs).wait()

    for s in range(NS):
        myrow = s * IDX_ROWS_PER_SCALE + t * NCHUNK
        def _group(g, carry):
            grow = myrow + g * IDXG
            pltpu.make_async_copy(col0.at[pl.ds(grow, IDXG)], idx0a,
                                  qp_a).start()
            pltpu.make_async_copy(col1.at[pl.ds(grow, IDXG)], idx1a,
                                  qs_a).start()
            pltpu.make_async_copy(col0s.at[c, pl.ds(grow, IDXG)], idx0s,
                                  qp_b).start()
            pltpu.make_async_copy(col1s.at[c, pl.ds(grow, IDXG)], idx1s,
                                  qs_b).start()
            pltpu.make_async_copy(col0.at[pl.ds(grow, IDXG)], idx0a,
                                  qp_a).wait()
            pltpu.make_async_copy(col1.at[pl.ds(grow, IDXG)], idx1a,
                                  qs_a).wait()
            pltpu.make_async_copy(col0s.at[c, pl.ds(grow, IDXG)], idx0s,
                                  qp_b).wait()
            pltpu.make_async_copy(col1s.at[c, pl.ds(grow, IDXG)], idx1s,
                                  qs_b).wait()
            _start(0, 0)

            def _pair(jj, carry2):
                for b in range(2):
                    cj = jj * 2 + b
                    _wait(b)

                    @pl.when(cj >= 1)
                    def _():
                        _wait_scatter(1 - b)

                    @pl.when(cj + 1 < IDXG)
                    def _():
                        _start(cj + 1, 1 - b)

                    _start_scatter(cj, b)
                return carry2
            lax.fori_loop(0, IDXG // 2, _pair, 0)
            _wait_scatter(1)
            return carry
        lax.fori_loop(0, NGROUP, _group, 0)
        plsc.subcore_barrier()

        pltpu.make_async_copy(macc0.at[pl.ds(r0, ROWS_PER_TILE)],
                              out.at[c, s, pl.ds(r0, ROWS_PER_TILE)],
                              rp_a).start()
        pltpu.make_async_copy(macc1.at[pl.ds(r0, ROWS_PER_TILE)],
                              out.at[c, NS + s, pl.ds(r0, ROWS_PER_TILE)],
                              rs_a).start()
        pltpu.make_async_copy(macc0.at[pl.ds(r0, ROWS_PER_TILE)],
                              out.at[c, s, pl.ds(r0, ROWS_PER_TILE)],
                              rp_a).wait()
        pltpu.make_async_copy(macc1.at[pl.ds(r0, ROWS_PER_TILE)],
                              out.at[c, NS + s, pl.ds(r0, ROWS_PER_TILE)],
                              rs_a).wait()
        for k in range(ROWS_PER_TILE // ZROWS):
            pltpu.make_async_copy(
                zbuf, macc0.at[pl.ds(r0 + k * ZROWS, ZROWS)], rp_b).start()
            pltpu.make_async_copy(
                zbuf, macc1.at[pl.ds(r0 + k * ZROWS, ZROWS)], rs_b).start()
        for k in range(ROWS_PER_TILE // ZROWS):
            pltpu.make_async_copy(
                zbuf, macc0.at[pl.ds(r0 + k * ZROWS, ZROWS)], rp_b).wait()
            pltpu.make_async_copy(
                zbuf, macc1.at[pl.ds(r0 + k * ZROWS, ZROWS)], rs_b).wait()
        plsc.subcore_barrier()


@functools.cache
def _make_sc_aggregate():
    mesh = plsc.VectorSubcoreMesh(core_axis_name="c", subcore_axis_name="s")
    return pl.kernel(
        _sc_body,
        out_type=jax.ShapeDtypeStruct((NC, NPAIR, N_PAD, H), jnp.float32),
        mesh=mesh,
        scratch_types=[
            pltpu.VMEM_SHARED((N_PAD, H), jnp.float32),
            pltpu.VMEM_SHARED((N_PAD, H), jnp.float32),
            pltpu.VMEM((IDXG, CHUNK), jnp.int32),
            pltpu.VMEM((IDXG, CHUNK), jnp.int32),
            pltpu.VMEM((IDXG, CHUNK), jnp.int32),
            pltpu.VMEM((IDXG, CHUNK), jnp.int32),
            pltpu.VMEM((CHUNK, H), jnp.float32),
            pltpu.VMEM((CHUNK, H), jnp.float32),
            pltpu.VMEM((CHUNK, H), jnp.float32),
            pltpu.VMEM((CHUNK, H), jnp.float32),
            pltpu.VMEM((ZROWS, H), jnp.float32),
            pltpu.SemaphoreType.DMA,
            pltpu.SemaphoreType.DMA,
            pltpu.SemaphoreType.DMA,
            pltpu.SemaphoreType.DMA,
            pltpu.SemaphoreType.DMA,
            pltpu.SemaphoreType.DMA,
            pltpu.SemaphoreType.DMA,
            pltpu.SemaphoreType.DMA,
        ],
        compiler_params=pltpu.CompilerParams(use_tc_tiling_on_sc=False),
    )


def _sc_aggregate(fsplit, col0, col1, col0s, col1s):
    f_cat = fsplit.reshape(NC * N_PAD, H)
    return _make_sc_aggregate()(f_cat, col0, col1, col0s, col1s)


def _dot(a, b):
    return lax.dot_general(a, b, (((1,), (0,)), ((), ())),
                           preferred_element_type=jnp.float32)


def _gn(x, w, b, eps=1e-5):
    mu = jnp.mean(x, axis=-1, keepdims=True)
    var = jnp.mean((x - mu) ** 2, axis=-1, keepdims=True)
    return (x - mu) / jnp.sqrt(var + eps) * w + b


R = 2000


def _input_body(c4_ref, w1c, b1c, w2c, gwc, gbc, w1s, b1s, w2s, gws, gbs,
                ffull_ref, fsplit_ref):
    c4 = c4_ref[...]
    centers = (c4[:, 0:2] + c4[:, 2:4]) * 0.5
    diff = c4[:, 2:4] - c4[:, 0:2]

    def mlp(x, w1t, b1, w2t, gw, gb):
        h = jnp.maximum(_dot(x, w1t[...]) + b1[...], 0.0)
        h = _dot(h, w2t[...])
        return _gn(h, gw[...], gb[...])

    f = jnp.maximum(mlp(centers, w1c, b1c, w2c, gwc, gbc)
                    + mlp(diff, w1s, b1s, w2s, gws, gbs), 0.0)
    ffull_ref[...] = f
    fsplit_ref[0] = f[:, :H]
    fsplit_ref[1] = f[:, H:]


def _block_body(ff_ref, m4_ref, wct_ref, wsub_ref, g1w, g1b, wlt_ref, g2w,
                g2b, fout_ref, fsplit_ref):
    f = ff_ref[...]
    m4 = m4_ref[...]
    temp = _dot(f, wct_ref[...])
    for c in range(NC):
        for p in range(NPAIR):
            temp = temp + _dot(m4[c, p], wsub_ref[c, p])
    t = jnp.maximum(_gn(temp, g1w[...], g1b[...]), 0.0)
    t = _gn(_dot(t, wlt_ref[...]), g2w[...], g2b[...])
    out = jnp.maximum(t + f, 0.0)
    fout_ref[...] = out
    fsplit_ref[0] = out[:, :H]
    fsplit_ref[1] = out[:, H:]


def _full_spec(shape):
    nd = len(shape)
    return pl.BlockSpec(shape, lambda i, _nd=nd: (0,) * _nd)


def _input_stage(coords4, w1c, b1c, w2c, gwc, gbc, w1s, b1s, w2s, gws, gbs):
    grid = N // R
    return pl.pallas_call(
        _input_body,
        grid=(grid,),
        in_specs=[
            pl.BlockSpec((R, 4), lambda i: (i, 0)),
            _full_spec((2, D)), _full_spec((1, D)), _full_spec((D, D)),
            _full_spec((1, D)), _full_spec((1, D)),
            _full_spec((2, D)), _full_spec((1, D)), _full_spec((D, D)),
            _full_spec((1, D)), _full_spec((1, D)),
        ],
        out_specs=[
            pl.BlockSpec((R, D), lambda i: (i, 0)),
            pl.BlockSpec((NC, R, H), lambda i: (0, i, 0)),
        ],
        out_shape=[
            jax.ShapeDtypeStruct((N, D), jnp.float32),
            jax.ShapeDtypeStruct((NC, N_PAD, H), jnp.float32),
        ],
    )(coords4, w1c, b1c, w2c, gwc, gbc, w1s, b1s, w2s, gws, gbs)


def _block_stage(ffull, m4, wct, wsub, g1w, g1b, wlt, g2w, g2b):
    grid = N // R
    return pl.pallas_call(
        _block_body,
        grid=(grid,),
        in_specs=[
            pl.BlockSpec((R, D), lambda i: (i, 0)),
            pl.BlockSpec((NC, NPAIR, R, H), lambda i: (0, 0, i, 0)),
            _full_spec((D, D)),
            _full_spec((NC, NPAIR, H, D)),
            _full_spec((1, D)), _full_spec((1, D)),
            _full_spec((D, D)),
            _full_spec((1, D)), _full_spec((1, D)),
        ],
        out_specs=[
            pl.BlockSpec((R, D), lambda i: (i, 0)),
            pl.BlockSpec((NC, R, H), lambda i: (0, i, 0)),
        ],
        out_shape=[
            jax.ShapeDtypeStruct((N, D), jnp.float32),
            jax.ShapeDtypeStruct((NC, N_PAD, H), jnp.float32),
        ],
    )(ffull, m4, wct, wsub, g1w, g1b, wlt, g2w, g2b)


def kernel(coords, conns, W_in1, b_in1, W_in2, gn_in_w, gn_in_b, W_seg1,
           b_seg1, W_seg2, gn_seg_w, gn_seg_b, W_center, W_pre, W_suc,
           gn1_w, gn1_b, W_lgn, gn2_w, gn2_b):
    coords4 = coords.reshape(N, 4)
    padv = (N + (jnp.arange(E_PAD - E, dtype=jnp.int32) % (N_PAD - N)))
    padv = jnp.broadcast_to(padv, (NS, E_PAD - E))
    col0 = jnp.concatenate([conns[1:, :, 0], padv], axis=1).reshape(-1, CHUNK)
    col1 = jnp.concatenate([conns[1:, :, 1], padv], axis=1).reshape(-1, CHUNK)
    col0s = jnp.stack([col0, col0 + N_PAD])
    col1s = jnp.stack([col1, col1 + N_PAD])
    r1 = lambda v: v.reshape(1, D)

    ffull, fsplit = _input_stage(
        coords4, W_in1.T, r1(b_in1), W_in2.T, r1(gn_in_w), r1(gn_in_b),
        W_seg1.T, r1(b_seg1), W_seg2.T, r1(gn_seg_w), r1(gn_seg_b))

    for i in range(NB):
        m4 = _sc_aggregate(fsplit, col0, col1, col0s, col1s)
        wt = jnp.swapaxes(jnp.concatenate([W_pre[i], W_suc[i]], axis=0), 1, 2)
        wsub = jnp.swapaxes(wt.reshape(NPAIR, NC, H, D), 0, 1)
        ffull, fsplit = _block_stage(
            ffull, m4, W_center[i].T, wsub, r1(gn1_w[i]), r1(gn1_b[i]),
            W_lgn[i].T, r1(gn2_w[i]), r1(gn2_b[i]))
    return ffull
